# baseline (speedup 1.0000x reference)
import functools

import jax
import jax.numpy as jnp
from jax import lax
from jax.experimental import pallas as pl
from jax.experimental.pallas import tpu as pltpu
from jax.experimental.pallas import tpu_sc as plsc

N = 10000
E = 320000
D = 128
FF = 128
G = 64
TW = 256

NP = 10240
EP = 327680
ECH = EP
NC = 2
NS = 16
NW = NC * NS
PER_W = ECH // NW
CH = 256
ITERS = PER_W // CH
CH4 = 2560


def _pack(t):
    lo = lax.bitcast_convert_type(t[:, :FF].astype(jnp.bfloat16), jnp.uint16)
    hi = lax.bitcast_convert_type(t[:, FF:].astype(jnp.bfloat16), jnp.uint16)
    return (hi.astype(jnp.int32) << 16) | lo.astype(jnp.int32)


def _unpack_lo(w):
    return lax.bitcast_convert_type(
        (w & 0xFFFF).astype(jnp.uint16), jnp.bfloat16).astype(jnp.float32)


def _unpack_hi(w):
    return lax.bitcast_convert_type(
        lax.shift_right_logical(w, 16).astype(jnp.uint16),
        jnp.bfloat16).astype(jnp.float32)


def _prep_body(xn_ref, wa_ref, wb_ref, ta_ref, tb_ref):
    xb = xn_ref[...]
    ta = jnp.dot(xb, wa_ref[...], preferred_element_type=jnp.float32)
    tb = jnp.dot(xb, wb_ref[...], preferred_element_type=jnp.float32)
    ta_ref[...] = _pack(ta)
    tb_ref[...] = _pack(tb)


def _prep(xnp, w1a, w1b):
    nb = 10
    rb = NP // nb
    return pl.pallas_call(
        _prep_body,
        grid=(nb,),
        in_specs=[
            pl.BlockSpec((rb, D), lambda i: (i, 0)),
            pl.BlockSpec((D, TW), lambda i: (0, 0)),
            pl.BlockSpec((D, TW), lambda i: (0, 0)),
        ],
        out_specs=[
            pl.BlockSpec((rb, FF), lambda i: (i, 0)),
            pl.BlockSpec((rb, FF), lambda i: (i, 0)),
        ],
        out_shape=[
            jax.ShapeDtypeStruct((NP, FF), jnp.int32),
            jax.ShapeDtypeStruct((NP, FF), jnp.int32),
        ],
    )(xnp, w1a, w1b)


def _gather_body(ta_hbm, tb_hbm, src_hbm, dst_hbm, ga_hbm, gb_hbm,
                 ia_v, ib_v, ba_v, bb_v, sa, sb):
    c = lax.axis_index("c")
    s = lax.axis_index("s")
    wid = s * NC + c
    base = wid * PER_W

    def it(i, carry):
        off = base + i * CH
        pltpu.sync_copy(src_hbm.at[pl.ds(off, CH)], ia_v)
        pltpu.sync_copy(dst_hbm.at[pl.ds(off, CH)], ib_v)
        da = pltpu.async_copy(ta_hbm.at[ia_v], ba_v, sa)
        db = pltpu.async_copy(tb_hbm.at[ib_v], bb_v, sb)
        da.wait()
        db.wait()
        pltpu.sync_copy(ba_v, ga_hbm.at[pl.ds(off, CH)])
        pltpu.sync_copy(bb_v, gb_hbm.at[pl.ds(off, CH)])
        return carry

    lax.fori_loop(0, ITERS, it, 0)


def _gather(ta, tb, src, dst):
    mesh = plsc.VectorSubcoreMesh(
        core_axis_name="c", subcore_axis_name="s",
        num_cores=NC, num_subcores=NS)
    f = functools.partial(
        pl.kernel,
        out_type=[
            jax.ShapeDtypeStruct((ECH, FF), jnp.int32),
            jax.ShapeDtypeStruct((ECH, FF), jnp.int32),
        ],
        mesh=mesh,
        scratch_types=[
            pltpu.VMEM((CH,), jnp.int32),
            pltpu.VMEM((CH,), jnp.int32),
            pltpu.VMEM((CH, FF), jnp.int32),
            pltpu.VMEM((CH, FF), jnp.int32),
            pltpu.SemaphoreType.DMA,
            pltpu.SemaphoreType.DMA,
        ],
        compiler_params=pltpu.CompilerParams(needs_layout_passes=False),
    )(_gather_body)
    return f(ta, tb, src, dst)


def _mlp_body(xe_ref, ga_ref, gb_ref, w1c_ref, b1_ref, w2_ref,
              b2_ref, s_ref):
    ga = ga_ref[...]
    gb = gb_ref[...]
    xw = jnp.dot(xe_ref[...].astype(jnp.bfloat16), w1c_ref[...],
                 preferred_element_type=jnp.float32)
    b1 = b1_ref[...]
    pre_e = _unpack_lo(ga) + _unpack_lo(gb) + xw[:, :FF] + b1[:, :FF]
    pre_f = _unpack_hi(ga) + _unpack_hi(gb) + xw[:, FF:] + b1[:, FF:]
    h = jnp.concatenate(
        [jnp.maximum(pre_e, 0.0), jnp.maximum(pre_f, 0.0)], axis=1)
    s = jnp.dot(h, w2_ref[...], preferred_element_type=jnp.float32)
    s_ref[...] = s + b2_ref[...]


def _mlp(xe, ga, gb, w1c, b1, w2blk, b2, n_edges, xe_blk_off):
    be = 512
    nb = n_edges // be
    off = xe_blk_off // be
    return pl.pallas_call(
        _mlp_body,
        grid=(nb,),
        in_specs=[
            pl.BlockSpec((be, D), lambda i: (i + off, 0)),
            pl.BlockSpec((be, FF), lambda i: (i, 0)),
            pl.BlockSpec((be, FF), lambda i: (i, 0)),
            pl.BlockSpec((D, TW), lambda i: (0, 0)),
            pl.BlockSpec((1, TW), lambda i: (0, 0)),
            pl.BlockSpec((TW, 2), lambda i: (0, 0)),
            pl.BlockSpec((1, 2), lambda i: (0, 0)),
        ],
        out_specs=pl.BlockSpec((be, 2), lambda i: (i, 0)),
        out_shape=jax.ShapeDtypeStruct((n_edges, 2), jnp.float32),
    )(xe, ga, gb, w1c, b1, w2blk, b2)


def _rsqrt(n2):
    yi = jnp.int32(0x5F3759DF) - lax.shift_right_logical(
        plsc.bitcast(n2, jnp.int32), jnp.int32(1))
    y = plsc.bitcast(yi, jnp.float32)
    for _ in range(3):
        t = n2 * y
        y = y * (1.5 - 0.5 * t * y)
    return y


def _scatter_body(ssf_hbm, src_hbm, dst_hbm, pos_hbm, zz_hbm,
                  out_hbm, ssf_v, se_v, isrc_v, idst_v, pos_v,
                  fx_v, fy_v, fz_v, acc_e, acc_x, acc_y, acc_z):
    c = lax.axis_index("c")
    s = lax.axis_index("s")
    wid = s * NC + c
    base = wid * PER_W

    @pl.when(s == 0)
    def _():
        pltpu.sync_copy(zz_hbm, acc_e)
        pltpu.sync_copy(zz_hbm, acc_x)
        pltpu.sync_copy(zz_hbm, acc_y)
        pltpu.sync_copy(zz_hbm, acc_z)

    pltpu.sync_copy(pos_hbm, pos_v)
    plsc.subcore_barrier()

    i2 = jnp.arange(16, dtype=jnp.int32) * 2

    def chunk(i, carry):
        off = base + i * CH4
        pltpu.sync_copy(ssf_hbm.at[pl.ds(2 * off, 2 * CH4)], ssf_v)
        pltpu.sync_copy(src_hbm.at[pl.ds(off, CH4)], isrc_v)
        pltpu.sync_copy(dst_hbm.at[pl.ds(off, CH4)], idst_v)

        def vec(j, carry2):
            sl = pl.ds(j * 16, 16)
            pair = i2 + j * 32
            vse = plsc.load_gather(ssf_v, [pair])
            vsf = plsc.load_gather(ssf_v, [pair + 1])
            vs = isrc_v[sl] * 3
            vd = idst_v[sl] * 3
            ax = plsc.load_gather(pos_v, [vs])
            ay = plsc.load_gather(pos_v, [vs + 1])
            az = plsc.load_gather(pos_v, [vs + 2])
            bx = plsc.load_gather(pos_v, [vd])
            by = plsc.load_gather(pos_v, [vd + 1])
            bz = plsc.load_gather(pos_v, [vd + 2])
            dx = ax - bx
            dy = ay - by
            dz = az - bz
            n2 = dx * dx + dy * dy + dz * dz
            w = vsf * _rsqrt(n2)
            se_v[sl] = vse
            fx_v[sl] = w * dx
            fy_v[sl] = w * dy
            fz_v[sl] = w * dz
            return carry2

        lax.fori_loop(0, CH4 // 16, vec, 0)
        pltpu.sync_copy(se_v, acc_e.at[isrc_v], add=True)
        pltpu.sync_copy(fx_v, acc_x.at[isrc_v], add=True)
        pltpu.sync_copy(fy_v, acc_y.at[isrc_v], add=True)
        pltpu.sync_copy(fz_v, acc_z.at[isrc_v], add=True)
        return carry

    lax.fori_loop(0, PER_W // CH4, chunk, 0)
    plsc.subcore_barrier()

    @pl.when(s == 0)
    def _():
        pltpu.sync_copy(acc_e, out_hbm.at[c, 0])
        pltpu.sync_copy(acc_x, out_hbm.at[c, 1])
        pltpu.sync_copy(acc_y, out_hbm.at[c, 2])
        pltpu.sync_copy(acc_z, out_hbm.at[c, 3])


def _scatter(ssf, src, dst, posf, zz):
    mesh = plsc.VectorSubcoreMesh(
        core_axis_name="c", subcore_axis_name="s",
        num_cores=NC, num_subcores=NS)
    f = functools.partial(
        pl.kernel,
        out_type=jax.ShapeDtypeStruct((NC, 4, NP), jnp.float32),
        mesh=mesh,
        scratch_types=[
            pltpu.VMEM((2 * CH4,), jnp.float32),
            pltpu.VMEM((CH4,), jnp.float32),
            pltpu.VMEM((CH4,), jnp.int32),
            pltpu.VMEM((CH4,), jnp.int32),
            pltpu.VMEM((3 * NP,), jnp.float32),
            pltpu.VMEM((CH4,), jnp.float32),
            pltpu.VMEM((CH4,), jnp.float32),
            pltpu.VMEM((CH4,), jnp.float32),
            pltpu.VMEM_SHARED((NP,), jnp.float32),
            pltpu.VMEM_SHARED((NP,), jnp.float32),
            pltpu.VMEM_SHARED((NP,), jnp.float32),
            pltpu.VMEM_SHARED((NP,), jnp.float32),
        ],
        compiler_params=pltpu.CompilerParams(needs_layout_passes=False),
    )(_scatter_body)
    return f(ssf, src, dst, posf, zz)


def _finish_body(pa_ref, batch_ref, en_ref, ft_ref):
    p = (pa_ref[0] + pa_ref[1])[:, :N]
    ft_ref[...] = p
    en = p[0:1, :]
    g = lax.broadcasted_iota(jnp.int32, (G, N), 0)
    onehot = (batch_ref[...] == g).astype(jnp.float32)
    en_ref[...] = lax.dot_general(
        onehot, en, (((1,), (1,)), ((), ())),
        preferred_element_type=jnp.float32)


def _finish(pa, batch2d):
    return pl.pallas_call(
        _finish_body,
        grid=(1,),
        in_specs=[
            pl.BlockSpec((NC, 4, NP), lambda i: (0, 0, 0)),
            pl.BlockSpec((1, N), lambda i: (0, 0)),
        ],
        out_specs=[
            pl.BlockSpec((G, 1), lambda i: (0, 0)),
            pl.BlockSpec((4, N), lambda i: (0, 0)),
        ],
        out_shape=[
            jax.ShapeDtypeStruct((G, 1), jnp.float32),
            jax.ShapeDtypeStruct((4, N), jnp.float32),
        ],
    )(pa, batch2d)


def kernel(x, pos, batch, edge_index, We1, be1, We2, be2, Wf1, bf1, Wf2, bf2):
    xn = x[:N]
    xe = x[N:]
    xnp = jnp.pad(xn, ((0, NP - N), (0, 0)))
    w1a = jnp.concatenate([We1[:D], Wf1[:D]], axis=1)
    w1b = jnp.concatenate([We1[D:2 * D], Wf1[D:2 * D]], axis=1)
    w1c = jnp.concatenate([We1[2 * D:], Wf1[2 * D:]],
                          axis=1).astype(jnp.bfloat16)
    b1 = jnp.concatenate([be1, bf1]).reshape(1, TW)
    b2 = jnp.stack([be2[0], bf2[0]]).reshape(1, 2)
    epad = EP - E
    srcp = jnp.concatenate([edge_index[0], jnp.full((epad,), N, jnp.int32)])
    dstp = jnp.concatenate([edge_index[1], jnp.full((epad,), N, jnp.int32)])
    posf = jnp.pad(pos.reshape(-1), (0, 3 * (NP - N)))
    zz = jnp.zeros((NP,), jnp.float32)
    w2blk = jnp.zeros((TW, 2), jnp.float32)
    w2blk = w2blk.at[:FF, 0].set(We2[:, 0]).at[FF:, 1].set(Wf2[:, 0])

    ta, tb = _prep(xnp, w1a, w1b)
    ga, gb = _gather(ta, tb, srcp, dstp)
    s2 = _mlp(xe, ga, gb, w1c, b1, w2blk, b2, E, 0)
    ssf = jnp.concatenate([s2, jnp.zeros((epad, 2), jnp.float32)]).reshape(-1)
    part = _scatter(ssf, srcp, dstp, posf, zz)
    energy, ft = _finish(part, batch.reshape(1, N))
    forces = ft[1:4].T
    return energy, forces


# speedup vs baseline: 1.6337x; 1.6337x over previous
import functools

import jax
import jax.numpy as jnp
from jax import lax
from jax.experimental import pallas as pl
from jax.experimental.pallas import tpu as pltpu
from jax.experimental.pallas import tpu_sc as plsc

N = 10000
E = 320000
D = 128
FF = 128
G = 64
TW = 256

NC = 2
NS = 16
NW = NC * NS
PER_W = E // NW
CH = 200
CH4 = 2000


def _pack(t):
    lo = lax.bitcast_convert_type(t[:, :FF].astype(jnp.bfloat16), jnp.uint16)
    hi = lax.bitcast_convert_type(t[:, FF:].astype(jnp.bfloat16), jnp.uint16)
    return (hi.astype(jnp.int32) << 16) | lo.astype(jnp.int32)


def _unpack_lo(w):
    return lax.bitcast_convert_type(
        (w & 0xFFFF).astype(jnp.uint16), jnp.bfloat16).astype(jnp.float32)


def _unpack_hi(w):
    return lax.bitcast_convert_type(
        lax.shift_right_logical(w, 16).astype(jnp.uint16),
        jnp.bfloat16).astype(jnp.float32)


def _prep_body(xn_ref, wa_ref, wb_ref, ta_ref, tb_ref):
    xb = xn_ref[...]
    ta = jnp.dot(xb, wa_ref[...], preferred_element_type=jnp.float32)
    tb = jnp.dot(xb, wb_ref[...], preferred_element_type=jnp.float32)
    ta_ref[...] = _pack(ta)
    tb_ref[...] = _pack(tb)


def _prep(xn, w1a, w1b):
    nb = 10
    rb = N // nb
    return pl.pallas_call(
        _prep_body,
        grid=(nb,),
        in_specs=[
            pl.BlockSpec((rb, D), lambda i: (i, 0)),
            pl.BlockSpec((D, TW), lambda i: (0, 0)),
            pl.BlockSpec((D, TW), lambda i: (0, 0)),
        ],
        out_specs=[
            pl.BlockSpec((rb, FF), lambda i: (i, 0)),
            pl.BlockSpec((rb, FF), lambda i: (i, 0)),
        ],
        out_shape=[
            jax.ShapeDtypeStruct((N, FF), jnp.int32),
            jax.ShapeDtypeStruct((N, FF), jnp.int32),
        ],
    )(xn, w1a, w1b)


def _gather_body(ta_hbm, tb_hbm, src_hbm, dst_hbm, ga_hbm, gb_hbm,
                 ia_v, ib_v, ba_v, bb_v, sa, sb):
    c = lax.axis_index("c")
    s = lax.axis_index("s")
    wid = s * NC + c
    base = wid * PER_W

    def it(i, carry):
        off = base + i * CH
        pltpu.sync_copy(src_hbm.at[pl.ds(off, CH)], ia_v)
        pltpu.sync_copy(dst_hbm.at[pl.ds(off, CH)], ib_v)
        da = pltpu.async_copy(ta_hbm.at[ia_v], ba_v, sa)
        db = pltpu.async_copy(tb_hbm.at[ib_v], bb_v, sb)
        da.wait()
        db.wait()
        pltpu.sync_copy(ba_v, ga_hbm.at[pl.ds(off, CH)])
        pltpu.sync_copy(bb_v, gb_hbm.at[pl.ds(off, CH)])
        return carry

    lax.fori_loop(0, PER_W // CH, it, 0)


def _gather(ta, tb, src, dst):
    mesh = plsc.VectorSubcoreMesh(
        core_axis_name="c", subcore_axis_name="s",
        num_cores=NC, num_subcores=NS)
    f = functools.partial(
        pl.kernel,
        out_type=[
            jax.ShapeDtypeStruct((E, FF), jnp.int32),
            jax.ShapeDtypeStruct((E, FF), jnp.int32),
        ],
        mesh=mesh,
        scratch_types=[
            pltpu.VMEM((CH,), jnp.int32),
            pltpu.VMEM((CH,), jnp.int32),
            pltpu.VMEM((CH, FF), jnp.int32),
            pltpu.VMEM((CH, FF), jnp.int32),
            pltpu.SemaphoreType.DMA,
            pltpu.SemaphoreType.DMA,
        ],
        compiler_params=pltpu.CompilerParams(needs_layout_passes=False),
    )(_gather_body)
    return f(ta, tb, src, dst)


def _mlp_body(xe_ref, ga_ref, gb_ref, w1c_ref, b1_ref, w2_ref,
              b2_ref, se_ref, sf_ref):
    ga = ga_ref[...]
    gb = gb_ref[...]
    xw = jnp.dot(xe_ref[...].astype(jnp.bfloat16), w1c_ref[...],
                 preferred_element_type=jnp.float32)
    b1 = b1_ref[...]
    pre_e = _unpack_lo(ga) + _unpack_lo(gb) + xw[:, :FF] + b1[:, :FF]
    pre_f = _unpack_hi(ga) + _unpack_hi(gb) + xw[:, FF:] + b1[:, FF:]
    h = jnp.concatenate(
        [jnp.maximum(pre_e, 0.0), jnp.maximum(pre_f, 0.0)], axis=1)
    s = jnp.dot(h, w2_ref[...], preferred_element_type=jnp.float32)
    s = s + b2_ref[...]
    se_ref[...] = s[:, 0]
    sf_ref[...] = s[:, 1]


def _mlp(xe, ga, gb, w1c, b1, w2blk, b2):
    be = 512
    nb = E // be
    return pl.pallas_call(
        _mlp_body,
        grid=(nb,),
        in_specs=[
            pl.BlockSpec((be, D), lambda i: (i, 0)),
            pl.BlockSpec((be, FF), lambda i: (i, 0)),
            pl.BlockSpec((be, FF), lambda i: (i, 0)),
            pl.BlockSpec((D, TW), lambda i: (0, 0)),
            pl.BlockSpec((1, TW), lambda i: (0, 0)),
            pl.BlockSpec((TW, 2), lambda i: (0, 0)),
            pl.BlockSpec((1, 2), lambda i: (0, 0)),
        ],
        out_specs=[
            pl.BlockSpec((be,), lambda i: (i,)),
            pl.BlockSpec((be,), lambda i: (i,)),
        ],
        out_shape=[
            jax.ShapeDtypeStruct((E,), jnp.float32),
            jax.ShapeDtypeStruct((E,), jnp.float32),
        ],
    )(xe, ga, gb, w1c, b1, w2blk, b2)


def _rsqrt(n2):
    yi = jnp.int32(0x5F3759DF) - lax.shift_right_logical(
        plsc.bitcast(n2, jnp.int32), jnp.int32(1))
    y = plsc.bitcast(yi, jnp.float32)
    for _ in range(3):
        t = n2 * y
        y = y * (1.5 - 0.5 * t * y)
    return y


def _scatter_body(se_hbm, sf_hbm, src_hbm, dst_hbm, pos_hbm, zz_hbm,
                  out_hbm, se_v, sf_v, isrc_v, idst_v, pos_v,
                  fx_v, fy_v, fz_v, acc_e, acc_x, acc_y, acc_z):
    c = lax.axis_index("c")
    s = lax.axis_index("s")
    wid = s * NC + c
    base = wid * PER_W

    @pl.when(s == 0)
    def _():
        pltpu.sync_copy(zz_hbm, acc_e)
        pltpu.sync_copy(zz_hbm, acc_x)
        pltpu.sync_copy(zz_hbm, acc_y)
        pltpu.sync_copy(zz_hbm, acc_z)

    pltpu.sync_copy(pos_hbm, pos_v)
    plsc.subcore_barrier()

    def chunk(i, carry):
        off = base + i * CH4
        pltpu.sync_copy(se_hbm.at[pl.ds(off, CH4)], se_v)
        pltpu.sync_copy(sf_hbm.at[pl.ds(off, CH4)], sf_v)
        pltpu.sync_copy(src_hbm.at[pl.ds(off, CH4)], isrc_v)
        pltpu.sync_copy(dst_hbm.at[pl.ds(off, CH4)], idst_v)

        def vec(j, carry2):
            sl = pl.ds(j * 16, 16)
            vsf = sf_v[sl]
            vs = isrc_v[sl] * 3
            vd = idst_v[sl] * 3
            ax = plsc.load_gather(pos_v, [vs])
            ay = plsc.load_gather(pos_v, [vs + 1])
            az = plsc.load_gather(pos_v, [vs + 2])
            bx = plsc.load_gather(pos_v, [vd])
            by = plsc.load_gather(pos_v, [vd + 1])
            bz = plsc.load_gather(pos_v, [vd + 2])
            dx = ax - bx
            dy = ay - by
            dz = az - bz
            n2 = dx * dx + dy * dy + dz * dz
            w = vsf * _rsqrt(n2)
            fx_v[sl] = w * dx
            fy_v[sl] = w * dy
            fz_v[sl] = w * dz
            return carry2

        lax.fori_loop(0, CH4 // 16, vec, 0)
        pltpu.sync_copy(se_v, acc_e.at[isrc_v], add=True)
        pltpu.sync_copy(fx_v, acc_x.at[isrc_v], add=True)
        pltpu.sync_copy(fy_v, acc_y.at[isrc_v], add=True)
        pltpu.sync_copy(fz_v, acc_z.at[isrc_v], add=True)
        return carry

    lax.fori_loop(0, PER_W // CH4, chunk, 0)
    plsc.subcore_barrier()

    @pl.when(s == 0)
    def _():
        pltpu.sync_copy(acc_e, out_hbm.at[c, 0])
        pltpu.sync_copy(acc_x, out_hbm.at[c, 1])
        pltpu.sync_copy(acc_y, out_hbm.at[c, 2])
        pltpu.sync_copy(acc_z, out_hbm.at[c, 3])


def _scatter(se, sf, src, dst, posf, zz):
    mesh = plsc.VectorSubcoreMesh(
        core_axis_name="c", subcore_axis_name="s",
        num_cores=NC, num_subcores=NS)
    f = functools.partial(
        pl.kernel,
        out_type=jax.ShapeDtypeStruct((NC, 4, N), jnp.float32),
        mesh=mesh,
        scratch_types=[
            pltpu.VMEM((CH4,), jnp.float32),
            pltpu.VMEM((CH4,), jnp.float32),
            pltpu.VMEM((CH4,), jnp.int32),
            pltpu.VMEM((CH4,), jnp.int32),
            pltpu.VMEM((3 * N,), jnp.float32),
            pltpu.VMEM((CH4,), jnp.float32),
            pltpu.VMEM((CH4,), jnp.float32),
            pltpu.VMEM((CH4,), jnp.float32),
            pltpu.VMEM_SHARED((N,), jnp.float32),
            pltpu.VMEM_SHARED((N,), jnp.float32),
            pltpu.VMEM_SHARED((N,), jnp.float32),
            pltpu.VMEM_SHARED((N,), jnp.float32),
        ],
        compiler_params=pltpu.CompilerParams(needs_layout_passes=False),
    )(_scatter_body)
    return f(se, sf, src, dst, posf, zz)


def _finish_body(pa_ref, batch_ref, en_ref, ft_ref):
    p = pa_ref[0] + pa_ref[1]
    ft_ref[...] = p
    en = p[0:1, :]
    g = lax.broadcasted_iota(jnp.int32, (G, N), 0)
    onehot = (batch_ref[...] == g).astype(jnp.float32)
    en_ref[...] = lax.dot_general(
        onehot, en, (((1,), (1,)), ((), ())),
        preferred_element_type=jnp.float32)


def _finish(pa, batch2d):
    return pl.pallas_call(
        _finish_body,
        grid=(1,),
        in_specs=[
            pl.BlockSpec((NC, 4, N), lambda i: (0, 0, 0)),
            pl.BlockSpec((1, N), lambda i: (0, 0)),
        ],
        out_specs=[
            pl.BlockSpec((G, 1), lambda i: (0, 0)),
            pl.BlockSpec((4, N), lambda i: (0, 0)),
        ],
        out_shape=[
            jax.ShapeDtypeStruct((G, 1), jnp.float32),
            jax.ShapeDtypeStruct((4, N), jnp.float32),
        ],
    )(pa, batch2d)


def kernel(x, pos, batch, edge_index, We1, be1, We2, be2, Wf1, bf1, Wf2, bf2):
    xn = x[:N]
    xe = x[N:]
    w1a = jnp.concatenate([We1[:D], Wf1[:D]], axis=1)
    w1b = jnp.concatenate([We1[D:2 * D], Wf1[D:2 * D]], axis=1)
    w1c = jnp.concatenate([We1[2 * D:], Wf1[2 * D:]],
                          axis=1).astype(jnp.bfloat16)
    b1 = jnp.concatenate([be1, bf1]).reshape(1, TW)
    b2 = jnp.stack([be2[0], bf2[0]]).reshape(1, 2)
    w2blk = jnp.zeros((TW, 2), jnp.float32)
    w2blk = w2blk.at[:FF, 0].set(We2[:, 0]).at[FF:, 1].set(Wf2[:, 0])
    src = edge_index[0]
    dst = edge_index[1]
    posf = pos.reshape(-1)
    zz = jnp.zeros((N,), jnp.float32)

    ta, tb = _prep(xn, w1a, w1b)
    ga, gb = _gather(ta, tb, src, dst)
    se, sf = _mlp(xe, ga, gb, w1c, b1, w2blk, b2)
    part = _scatter(se, sf, src, dst, posf, zz)
    energy, ft = _finish(part, batch.reshape(1, N))
    forces = ft[1:4].T
    return energy, forces


# speedup vs baseline: 1.6688x; 1.0214x over previous
import functools

import jax
import jax.numpy as jnp
from jax import lax
from jax.experimental import pallas as pl
from jax.experimental.pallas import tpu as pltpu
from jax.experimental.pallas import tpu_sc as plsc

N = 10000
E = 320000
D = 128
FF = 128
G = 64
TW = 256

NC = 2
NS = 16
NW = NC * NS
PER_W = E // NW
CH = 400
CH4 = 2000


def _pack(t):
    lo = lax.bitcast_convert_type(t[:, :FF].astype(jnp.bfloat16), jnp.uint16)
    hi = lax.bitcast_convert_type(t[:, FF:].astype(jnp.bfloat16), jnp.uint16)
    return (hi.astype(jnp.int32) << 16) | lo.astype(jnp.int32)


def _unpack_lo(w):
    return lax.bitcast_convert_type(
        (w & 0xFFFF).astype(jnp.uint16), jnp.bfloat16).astype(jnp.float32)


def _unpack_hi(w):
    return lax.bitcast_convert_type(
        lax.shift_right_logical(w, 16).astype(jnp.uint16),
        jnp.bfloat16).astype(jnp.float32)


def _prep_body(xn_ref, wa_ref, wb_ref, ta_ref, tb_ref):
    xb = xn_ref[...]
    ta = jnp.dot(xb, wa_ref[...], preferred_element_type=jnp.float32)
    tb = jnp.dot(xb, wb_ref[...], preferred_element_type=jnp.float32)
    ta_ref[...] = _pack(ta)
    tb_ref[...] = _pack(tb)


def _prep(xn, w1a, w1b):
    nb = 10
    rb = N // nb
    return pl.pallas_call(
        _prep_body,
        grid=(nb,),
        in_specs=[
            pl.BlockSpec((rb, D), lambda i: (i, 0)),
            pl.BlockSpec((D, TW), lambda i: (0, 0)),
            pl.BlockSpec((D, TW), lambda i: (0, 0)),
        ],
        out_specs=[
            pl.BlockSpec((rb, FF), lambda i: (i, 0)),
            pl.BlockSpec((rb, FF), lambda i: (i, 0)),
        ],
        out_shape=[
            jax.ShapeDtypeStruct((N, FF), jnp.int32),
            jax.ShapeDtypeStruct((N, FF), jnp.int32),
        ],
    )(xn, w1a, w1b)


def _gather_body(ta_hbm, tb_hbm, src_hbm, dst_hbm, ga_hbm, gb_hbm,
                 ia_v, ib_v, ba_v, bb_v, sa, sb):
    c = lax.axis_index("c")
    s = lax.axis_index("s")
    wid = s * NC + c
    base = wid * PER_W

    def it(i, carry):
        off = base + i * CH
        pltpu.sync_copy(src_hbm.at[pl.ds(off, CH)], ia_v)
        pltpu.sync_copy(dst_hbm.at[pl.ds(off, CH)], ib_v)
        da = pltpu.async_copy(ta_hbm.at[ia_v], ba_v, sa)
        db = pltpu.async_copy(tb_hbm.at[ib_v], bb_v, sb)
        da.wait()
        db.wait()
        pltpu.sync_copy(ba_v, ga_hbm.at[pl.ds(off, CH)])
        pltpu.sync_copy(bb_v, gb_hbm.at[pl.ds(off, CH)])
        return carry

    lax.fori_loop(0, PER_W // CH, it, 0)


def _gather(ta, tb, src, dst):
    mesh = plsc.VectorSubcoreMesh(
        core_axis_name="c", subcore_axis_name="s",
        num_cores=NC, num_subcores=NS)
    f = functools.partial(
        pl.kernel,
        out_type=[
            jax.ShapeDtypeStruct((E, FF), jnp.int32),
            jax.ShapeDtypeStruct((E, FF), jnp.int32),
        ],
        mesh=mesh,
        scratch_types=[
            pltpu.VMEM((CH,), jnp.int32),
            pltpu.VMEM((CH,), jnp.int32),
            pltpu.VMEM((CH, FF), jnp.int32),
            pltpu.VMEM((CH, FF), jnp.int32),
            pltpu.SemaphoreType.DMA,
            pltpu.SemaphoreType.DMA,
        ],
        compiler_params=pltpu.CompilerParams(needs_layout_passes=False),
    )(_gather_body)
    return f(ta, tb, src, dst)


def _mlp_body(xe_ref, ga_ref, gb_ref, w1c_ref, b1_ref, w2_ref,
              b2_ref, se_ref, sf_ref):
    ga = ga_ref[...]
    gb = gb_ref[...]
    xw = jnp.dot(xe_ref[...].astype(jnp.bfloat16), w1c_ref[...],
                 preferred_element_type=jnp.float32)
    b1 = b1_ref[...]
    pre_e = _unpack_lo(ga) + _unpack_lo(gb) + xw[:, :FF] + b1[:, :FF]
    pre_f = _unpack_hi(ga) + _unpack_hi(gb) + xw[:, FF:] + b1[:, FF:]
    h = jnp.concatenate(
        [jnp.maximum(pre_e, 0.0), jnp.maximum(pre_f, 0.0)], axis=1)
    s = jnp.dot(h, w2_ref[...], preferred_element_type=jnp.float32)
    s = s + b2_ref[...]
    se_ref[...] = s[:, 0]
    sf_ref[...] = s[:, 1]


def _mlp(xe, ga, gb, w1c, b1, w2blk, b2):
    be = 512
    nb = E // be
    return pl.pallas_call(
        _mlp_body,
        grid=(nb,),
        in_specs=[
            pl.BlockSpec((be, D), lambda i: (i, 0)),
            pl.BlockSpec((be, FF), lambda i: (i, 0)),
            pl.BlockSpec((be, FF), lambda i: (i, 0)),
            pl.BlockSpec((D, TW), lambda i: (0, 0)),
            pl.BlockSpec((1, TW), lambda i: (0, 0)),
            pl.BlockSpec((TW, 2), lambda i: (0, 0)),
            pl.BlockSpec((1, 2), lambda i: (0, 0)),
        ],
        out_specs=[
            pl.BlockSpec((be,), lambda i: (i,)),
            pl.BlockSpec((be,), lambda i: (i,)),
        ],
        out_shape=[
            jax.ShapeDtypeStruct((E,), jnp.float32),
            jax.ShapeDtypeStruct((E,), jnp.float32),
        ],
    )(xe, ga, gb, w1c, b1, w2blk, b2)


def _rsqrt(n2):
    yi = jnp.int32(0x5F3759DF) - lax.shift_right_logical(
        plsc.bitcast(n2, jnp.int32), jnp.int32(1))
    y = plsc.bitcast(yi, jnp.float32)
    for _ in range(3):
        t = n2 * y
        y = y * (1.5 - 0.5 * t * y)
    return y


def _scatter_body(se_hbm, sf_hbm, src_hbm, dst_hbm, pos_hbm, zz_hbm,
                  out_hbm, se_v, sf_v, isrc_v, idst_v, pos_v,
                  fx_v, fy_v, fz_v, acc_e, acc_x, acc_y, acc_z):
    c = lax.axis_index("c")
    s = lax.axis_index("s")
    wid = s * NC + c
    base = wid * PER_W

    @pl.when(s == 0)
    def _():
        pltpu.sync_copy(zz_hbm, acc_e)
        pltpu.sync_copy(zz_hbm, acc_x)
        pltpu.sync_copy(zz_hbm, acc_y)
        pltpu.sync_copy(zz_hbm, acc_z)

    pltpu.sync_copy(pos_hbm, pos_v)
    plsc.subcore_barrier()

    def chunk(i, carry):
        off = base + i * CH4
        pltpu.sync_copy(se_hbm.at[pl.ds(off, CH4)], se_v)
        pltpu.sync_copy(sf_hbm.at[pl.ds(off, CH4)], sf_v)
        pltpu.sync_copy(src_hbm.at[pl.ds(off, CH4)], isrc_v)
        pltpu.sync_copy(dst_hbm.at[pl.ds(off, CH4)], idst_v)

        def vec(j, carry2):
            sl = pl.ds(j * 16, 16)
            vsf = sf_v[sl]
            vs = isrc_v[sl] * 3
            vd = idst_v[sl] * 3
            ax = plsc.load_gather(pos_v, [vs])
            ay = plsc.load_gather(pos_v, [vs + 1])
            az = plsc.load_gather(pos_v, [vs + 2])
            bx = plsc.load_gather(pos_v, [vd])
            by = plsc.load_gather(pos_v, [vd + 1])
            bz = plsc.load_gather(pos_v, [vd + 2])
            dx = ax - bx
            dy = ay - by
            dz = az - bz
            n2 = dx * dx + dy * dy + dz * dz
            w = vsf * _rsqrt(n2)
            fx_v[sl] = w * dx
            fy_v[sl] = w * dy
            fz_v[sl] = w * dz
            return carry2

        lax.fori_loop(0, CH4 // 16, vec, 0)
        pltpu.sync_copy(se_v, acc_e.at[isrc_v], add=True)
        pltpu.sync_copy(fx_v, acc_x.at[isrc_v], add=True)
        pltpu.sync_copy(fy_v, acc_y.at[isrc_v], add=True)
        pltpu.sync_copy(fz_v, acc_z.at[isrc_v], add=True)
        return carry

    lax.fori_loop(0, PER_W // CH4, chunk, 0)
    plsc.subcore_barrier()

    @pl.when(s == 0)
    def _():
        pltpu.sync_copy(acc_e, out_hbm.at[c, 0])
        pltpu.sync_copy(acc_x, out_hbm.at[c, 1])
        pltpu.sync_copy(acc_y, out_hbm.at[c, 2])
        pltpu.sync_copy(acc_z, out_hbm.at[c, 3])


def _scatter(se, sf, src, dst, posf, zz):
    mesh = plsc.VectorSubcoreMesh(
        core_axis_name="c", subcore_axis_name="s",
        num_cores=NC, num_subcores=NS)
    f = functools.partial(
        pl.kernel,
        out_type=jax.ShapeDtypeStruct((NC, 4, N), jnp.float32),
        mesh=mesh,
        scratch_types=[
            pltpu.VMEM((CH4,), jnp.float32),
            pltpu.VMEM((CH4,), jnp.float32),
            pltpu.VMEM((CH4,), jnp.int32),
            pltpu.VMEM((CH4,), jnp.int32),
            pltpu.VMEM((3 * N,), jnp.float32),
            pltpu.VMEM((CH4,), jnp.float32),
            pltpu.VMEM((CH4,), jnp.float32),
            pltpu.VMEM((CH4,), jnp.float32),
            pltpu.VMEM_SHARED((N,), jnp.float32),
            pltpu.VMEM_SHARED((N,), jnp.float32),
            pltpu.VMEM_SHARED((N,), jnp.float32),
            pltpu.VMEM_SHARED((N,), jnp.float32),
        ],
        compiler_params=pltpu.CompilerParams(needs_layout_passes=False),
    )(_scatter_body)
    return f(se, sf, src, dst, posf, zz)


def _finish_body(pa_ref, batch_ref, en_ref, ft_ref):
    p = pa_ref[0] + pa_ref[1]
    ft_ref[...] = p
    en = p[0:1, :]
    g = lax.broadcasted_iota(jnp.int32, (G, N), 0)
    onehot = (batch_ref[...] == g).astype(jnp.float32)
    en_ref[...] = lax.dot_general(
        onehot, en, (((1,), (1,)), ((), ())),
        preferred_element_type=jnp.float32)


def _finish(pa, batch2d):
    return pl.pallas_call(
        _finish_body,
        grid=(1,),
        in_specs=[
            pl.BlockSpec((NC, 4, N), lambda i: (0, 0, 0)),
            pl.BlockSpec((1, N), lambda i: (0, 0)),
        ],
        out_specs=[
            pl.BlockSpec((G, 1), lambda i: (0, 0)),
            pl.BlockSpec((4, N), lambda i: (0, 0)),
        ],
        out_shape=[
            jax.ShapeDtypeStruct((G, 1), jnp.float32),
            jax.ShapeDtypeStruct((4, N), jnp.float32),
        ],
    )(pa, batch2d)


def kernel(x, pos, batch, edge_index, We1, be1, We2, be2, Wf1, bf1, Wf2, bf2):
    xn = x[:N]
    xe = x[N:]
    w1a = jnp.concatenate([We1[:D], Wf1[:D]], axis=1)
    w1b = jnp.concatenate([We1[D:2 * D], Wf1[D:2 * D]], axis=1)
    w1c = jnp.concatenate([We1[2 * D:], Wf1[2 * D:]],
                          axis=1).astype(jnp.bfloat16)
    b1 = jnp.concatenate([be1, bf1]).reshape(1, TW)
    b2 = jnp.stack([be2[0], bf2[0]]).reshape(1, 2)
    w2blk = jnp.zeros((TW, 2), jnp.float32)
    w2blk = w2blk.at[:FF, 0].set(We2[:, 0]).at[FF:, 1].set(Wf2[:, 0])
    src = edge_index[0]
    dst = edge_index[1]
    posf = pos.reshape(-1)
    zz = jnp.zeros((N,), jnp.float32)

    ta, tb = _prep(xn, w1a, w1b)
    ga, gb = _gather(ta, tb, src, dst)
    se, sf = _mlp(xe, ga, gb, w1c, b1, w2blk, b2)
    part = _scatter(se, sf, src, dst, posf, zz)
    energy, ft = _finish(part, batch.reshape(1, N))
    forces = ft[1:4].T
    return energy, forces
